# baseline (speedup 1.0000x reference)
import functools

import jax
import jax.numpy as jnp
from jax import lax
from jax.experimental import pallas as pl
from jax.experimental.pallas import tpu as pltpu
from jax.experimental.pallas import tpu_sc as plsc


def _topk_idx_kernel(xpad_ref, xt_ref, kv_ref, idx_ref, w_ref, d2_ref,
                     ia_ref, *, n, k_max):
    bi = pl.program_id(0)
    kv = kv_ref[...]
    kv_max = jnp.max(kv, axis=1, keepdims=True)
    lane = lax.broadcasted_iota(jnp.int32, kv.shape, 1)
    kidx = jnp.min(jnp.where(kv == kv_max, lane, kv.shape[1]),
                   axis=1, keepdims=True)
    k = kidx[0, 0] + 1
    inv_k = 1.0 / k.astype(jnp.float32)
    w_ref[...] = jnp.where(lane < k, inv_k, 0.0)

    xp = xpad_ref[0]
    xt = xt_ref[0]

    a2 = jnp.sum(xp * xp, axis=1, keepdims=True)
    b2 = jnp.sum(xt * xt, axis=0, keepdims=True)
    xpb = xp.astype(jnp.bfloat16).astype(jnp.float32)
    xtb = xt.astype(jnp.bfloat16).astype(jnp.float32)
    cross = xpb[:, 0:1] * xtb[0:1, :]
    cross = cross + xpb[:, 1:2] * xtb[1:2, :]
    cross = cross + xpb[:, 2:3] * xtb[2:3, :]
    d2_ref[...] = jnp.maximum(a2 + b2 - 2.0 * cross, 0.0)

    cidx = lax.broadcasted_iota(jnp.int32, (d2_ref.shape[0], n), 1)
    jlane = lax.broadcasted_iota(jnp.int32, ia_ref.shape, 1)
    base = bi * n
    ia_ref[...] = jnp.zeros_like(ia_ref)

    unroll = 4

    def chunk_body(c, _):
        for i in range(unroll):
            j = c * unroll + i
            d2v = d2_ref[...]
            m = jnp.min(d2v, axis=1, keepdims=True)
            eq = d2v == m
            first = jnp.min(jnp.where(eq, cidx, n), axis=1, keepdims=True)
            ia_ref[...] = jnp.where(jlane == j, first + base, ia_ref[...])
            d2_ref[...] = jnp.where(cidx == first, jnp.inf, d2v)
        return 0

    lax.fori_loop(0, (k + unroll - 1) // unroll, chunk_body, 0)
    idx_ref[0] = ia_ref[:, :k_max]


def _make_sc_gather(bn, f, k_max, rows_per_chunk, nc, ns):
    nw = nc * ns
    pts_per_w = bn // nw
    idx_rows_per_w = pts_per_w * k_max // 128
    chunks = idx_rows_per_w // rows_per_chunk
    pts_per_chunk = rows_per_chunk * 128 // k_max
    mesh = plsc.VectorSubcoreMesh(core_axis_name="c", subcore_axis_name="s")

    @functools.partial(
        pl.kernel, mesh=mesh,
        compiler_params=pltpu.CompilerParams(use_tc_tiling_on_sc=False),
        out_type=jax.ShapeDtypeStruct((bn, f), jnp.float32),
        scratch_types=[
            pltpu.VMEM((128,), jnp.float32),
            pltpu.VMEM((idx_rows_per_w, 128), jnp.int32),
            pltpu.VMEM((rows_per_chunk * 128, f), jnp.float32),
            pltpu.VMEM((pts_per_chunk, f), jnp.float32),
            pltpu.SemaphoreType.DMA,
        ],
    )
    def sc_gather(idx_hbm, preds_hbm, w_hbm, out_hbm,
                  w_v, idx_v, rows_v, out_v, sem):
        wid = lax.axis_index("s") * nc + lax.axis_index("c")
        pltpu.sync_copy(w_hbm, w_v)
        wvecs = [w_v[pl.ds(16 * i, 16)] for i in range((k_max + 15) // 16)]
        wj = [wvecs[j // 16][j % 16] for j in range(k_max)]
        pltpu.sync_copy(idx_hbm.at[pl.ds(wid * idx_rows_per_w,
                                         idx_rows_per_w)], idx_v)

        def chunk_body(ci, _):
            copies = [
                pltpu.async_copy(
                    preds_hbm.at[idx_v.at[ci * rows_per_chunk + g]],
                    rows_v.at[pl.ds(g * 128, 128)], sem)
                for g in range(rows_per_chunk)
            ]
            for cp in copies:
                cp.wait()

            def point_body(p, _):
                rbase = p * k_max
                for fc in range(f // 16):
                    acc = jnp.zeros((16,), jnp.float32)
                    for j in range(k_max):
                        acc = acc + wj[j] * rows_v[rbase + j,
                                                   pl.ds(fc * 16, 16)]
                    out_v[p, pl.ds(fc * 16, 16)] = acc
                return 0

            lax.fori_loop(0, pts_per_chunk, point_body, 0)
            p0 = wid * pts_per_w + ci * pts_per_chunk
            pltpu.sync_copy(out_v, out_hbm.at[pl.ds(p0, pts_per_chunk)])
            return 0

        lax.fori_loop(0, chunks, chunk_body, 0)

    return sc_gather


def kernel(x, preds, k_vector):
    b, n, _ = x.shape
    f = preds.shape[-1]
    k_max = k_vector.shape[0]
    r = min(256, n)
    bn = b * n

    xpad = jnp.pad(x, ((0, 0), (0, 0), (0, 128 - x.shape[-1])))
    xt = jnp.pad(jnp.transpose(x, (0, 2, 1)),
                 ((0, 0), (0, 8 - x.shape[-1]), (0, 0)))
    kv = jnp.full((1, 128), -jnp.inf, dtype=k_vector.dtype)
    kv = lax.dynamic_update_slice(kv, k_vector[None, :], (0, 0))

    grid = (b, n // r)
    idx, w = pl.pallas_call(
        functools.partial(_topk_idx_kernel, n=n, k_max=k_max),
        grid=grid,
        in_specs=[
            pl.BlockSpec((1, r, 128), lambda bi, i: (bi, i, 0)),
            pl.BlockSpec((1, 8, n), lambda bi, i: (bi, 0, 0)),
            pl.BlockSpec((1, 128), lambda bi, i: (0, 0)),
        ],
        out_specs=[
            pl.BlockSpec((1, r, k_max), lambda bi, i: (bi, i, 0)),
            pl.BlockSpec((1, 128), lambda bi, i: (0, 0)),
        ],
        out_shape=[
            jax.ShapeDtypeStruct((b, n, k_max), jnp.int32),
            jax.ShapeDtypeStruct((1, 128), jnp.float32),
        ],
        scratch_shapes=[
            pltpu.VMEM((r, n), jnp.float32),
            pltpu.VMEM((r, 128), jnp.int32),
        ],
    )(xpad, xt, kv)

    info = plsc.get_sparse_core_info()
    nc, ns = info.num_cores, info.num_subcores
    idx2 = idx.reshape(bn * k_max // 128, 128)
    preds_flat = preds.reshape(bn, f)
    sc = _make_sc_gather(bn, f, k_max, rows_per_chunk=5, nc=nc, ns=ns)
    out_flat = sc(idx2, preds_flat, w.reshape(128))
    return out_flat.reshape(b, n, f)


# speedup vs baseline: 2.7753x; 2.7753x over previous
import functools

import jax
import jax.numpy as jnp
from jax import lax
from jax.experimental import pallas as pl
from jax.experimental.pallas import tpu as pltpu
from jax.experimental.pallas import tpu_sc as plsc


def _topk_idx_kernel(xpad_ref, xt_ref, kv_ref, idx_ref, w_ref, d2_ref,
                     *, n, k_max):
    bi = pl.program_id(0)
    kv = kv_ref[...]
    kv_max = jnp.max(kv, axis=1, keepdims=True)
    lane = lax.broadcasted_iota(jnp.int32, kv.shape, 1)
    kidx = jnp.min(jnp.where(kv == kv_max, lane, kv.shape[1]),
                   axis=1, keepdims=True)
    k = kidx[0, 0] + 1
    inv_k = 1.0 / k.astype(jnp.float32)
    w_ref[...] = jnp.where(lane < k, inv_k, 0.0)

    xp = xpad_ref[0]
    xt = xt_ref[0]

    a2 = jnp.sum(xp * xp, axis=1, keepdims=True)
    b2 = jnp.sum(xt * xt, axis=0, keepdims=True)
    xpb = xp.astype(jnp.bfloat16).astype(jnp.float32)
    xtb = xt.astype(jnp.bfloat16).astype(jnp.float32)
    cross = xpb[:, 0:1] * xtb[0:1, :]
    cross = cross + xpb[:, 1:2] * xtb[1:2, :]
    cross = cross + xpb[:, 2:3] * xtb[2:3, :]
    d2_ref[...] = jnp.maximum(a2 + b2 - 2.0 * cross, 0.0)

    cidx = lax.broadcasted_iota(
        jnp.int32, (d2_ref.shape[0], n), 1).astype(jnp.float32)
    nf = jnp.float32(n)
    base = bi * n
    for j in range(k_max):
        d2v = d2_ref[...]
        m = jnp.min(d2v, axis=1, keepdims=True)
        eq = d2v == m
        first = jnp.min(jnp.where(eq, cidx, nf), axis=1, keepdims=True)
        idx_ref[0, :, j:j + 1] = first.astype(jnp.int32) + base
        d2_ref[...] = jnp.where(cidx == first, jnp.inf, d2v)


def _make_sc_gather(bn, f, k_max, rows_per_chunk, nc, ns):
    nw = nc * ns
    pts_per_w = bn // nw
    idx_rows_per_w = pts_per_w * k_max // 128
    chunks = idx_rows_per_w // rows_per_chunk
    pts_per_chunk = rows_per_chunk * 128 // k_max
    mesh = plsc.VectorSubcoreMesh(core_axis_name="c", subcore_axis_name="s")

    @functools.partial(
        pl.kernel, mesh=mesh,
        compiler_params=pltpu.CompilerParams(use_tc_tiling_on_sc=False),
        out_type=jax.ShapeDtypeStruct((bn, f), jnp.float32),
        scratch_types=[
            pltpu.VMEM((128,), jnp.float32),
            pltpu.VMEM((idx_rows_per_w, 128), jnp.int32),
            pltpu.VMEM((rows_per_chunk * 128, f), jnp.float32),
            pltpu.VMEM((pts_per_chunk, f), jnp.float32),
            pltpu.SemaphoreType.DMA,
        ],
    )
    def sc_gather(idx_hbm, preds_hbm, w_hbm, out_hbm,
                  w_v, idx_v, rows_v, out_v, sem):
        wid = lax.axis_index("s") * nc + lax.axis_index("c")
        pltpu.sync_copy(w_hbm, w_v)
        wvecs = [w_v[pl.ds(16 * i, 16)] for i in range((k_max + 15) // 16)]
        wj = [wvecs[j // 16][j % 16] for j in range(k_max)]
        pltpu.sync_copy(idx_hbm.at[pl.ds(wid * idx_rows_per_w,
                                         idx_rows_per_w)], idx_v)

        def chunk_body(ci, _):
            copies = [
                pltpu.async_copy(
                    preds_hbm.at[idx_v.at[ci * rows_per_chunk + g]],
                    rows_v.at[pl.ds(g * 128, 128)], sem)
                for g in range(rows_per_chunk)
            ]
            for cp in copies:
                cp.wait()

            def point_body(p, _):
                rbase = p * k_max
                for fc in range(f // 16):
                    acc = jnp.zeros((16,), jnp.float32)
                    for j in range(k_max):
                        acc = acc + wj[j] * rows_v[rbase + j,
                                                   pl.ds(fc * 16, 16)]
                    out_v[p, pl.ds(fc * 16, 16)] = acc
                return 0

            lax.fori_loop(0, pts_per_chunk, point_body, 0)
            p0 = wid * pts_per_w + ci * pts_per_chunk
            pltpu.sync_copy(out_v, out_hbm.at[pl.ds(p0, pts_per_chunk)])
            return 0

        lax.fori_loop(0, chunks, chunk_body, 0)

    return sc_gather


def kernel(x, preds, k_vector):
    b, n, _ = x.shape
    f = preds.shape[-1]
    k_max = k_vector.shape[0]
    r = min(256, n)
    bn = b * n

    xpad = jnp.pad(x, ((0, 0), (0, 0), (0, 128 - x.shape[-1])))
    xt = jnp.pad(jnp.transpose(x, (0, 2, 1)),
                 ((0, 0), (0, 8 - x.shape[-1]), (0, 0)))
    kv = jnp.full((1, 128), -jnp.inf, dtype=k_vector.dtype)
    kv = lax.dynamic_update_slice(kv, k_vector[None, :], (0, 0))

    grid = (b, n // r)
    idx, w = pl.pallas_call(
        functools.partial(_topk_idx_kernel, n=n, k_max=k_max),
        grid=grid,
        in_specs=[
            pl.BlockSpec((1, r, 128), lambda bi, i: (bi, i, 0)),
            pl.BlockSpec((1, 8, n), lambda bi, i: (bi, 0, 0)),
            pl.BlockSpec((1, 128), lambda bi, i: (0, 0)),
        ],
        out_specs=[
            pl.BlockSpec((1, r, k_max), lambda bi, i: (bi, i, 0)),
            pl.BlockSpec((1, 128), lambda bi, i: (0, 0)),
        ],
        out_shape=[
            jax.ShapeDtypeStruct((b, n, k_max), jnp.int32),
            jax.ShapeDtypeStruct((1, 128), jnp.float32),
        ],
        scratch_shapes=[
            pltpu.VMEM((r, n), jnp.float32),
        ],
    )(xpad, xt, kv)

    info = plsc.get_sparse_core_info()
    nc, ns = info.num_cores, info.num_subcores
    idx2 = idx.reshape(bn * k_max // 128, 128)
    preds_flat = preds.reshape(bn, f)
    sc = _make_sc_gather(bn, f, k_max, rows_per_chunk=5, nc=nc, ns=ns)
    out_flat = sc(idx2, preds_flat, w.reshape(128))
    return out_flat.reshape(b, n, f)
